# baseline (speedup 1.0000x reference)
import jax
import jax.numpy as jnp
from jax.experimental import pallas as pl
from jax.experimental.pallas import tpu as pltpu


def _gru_step(x, h, wi_ref, wh_ref, bi_ref, bh_ref):
    bf = jnp.bfloat16
    gi = jnp.dot(x.astype(bf), wi_ref[...], preferred_element_type=jnp.float32) + bi_ref[...]
    gh = jnp.dot(h.astype(bf), wh_ref[...], preferred_element_type=jnp.float32) + bh_ref[...]
    H = h.shape[1]
    i_r, i_z, i_n = gi[:, :H], gi[:, H : 2 * H], gi[:, 2 * H :]
    h_r, h_z, h_n = gh[:, :H], gh[:, H : 2 * H], gh[:, 2 * H :]
    r = jax.nn.sigmoid(i_r + h_r)
    z = jax.nn.sigmoid(i_z + h_z)
    n = jnp.tanh(i_n + r * h_n)
    return (1.0 - z) * n + z * h


def _logsoftmax_nll(h, proj_ref, tgt):
    logits = jnp.dot(h.astype(jnp.bfloat16), proj_ref[...],
                     preferred_element_type=jnp.float32)
    m = jnp.max(logits, axis=-1, keepdims=True)
    lse = jnp.log(jnp.sum(jnp.exp(logits - m), axis=-1, keepdims=True)) + m
    logp = logits - lse
    col = jax.lax.broadcasted_iota(jnp.int32, logits.shape, 1)
    picked = jnp.sum(jnp.where(col == tgt, logp, 0.0), axis=-1, keepdims=True)
    return -picked


def _fused_kernel(
    emb_conf_ref,
    trg_emb_ref,
    src_emb_ref,
    c_wi, c_wh, c_bi, c_bh,
    d_wi, d_wh, d_bi, d_bh,
    e_wi, e_wh, e_bi, e_bh,
    s_wi, s_wh, s_bi, s_bh,
    wm, bm, wv, bv, wl2h, bl2h,
    conf_proj, proj_head, sd_proj,
    adj_ref,
    trg_sub,
    src_sub,
    sd_sub,
    conf_out,
    nll_out,
    kl_out,
    sd_out,
    h_conf, h_dec,
):
    t = pl.program_id(0)
    B, H = h_conf.shape
    L = adj_ref.shape[0]

    @pl.when(t == 0)
    def _init():
        h = jnp.zeros((B, H), jnp.float32)
        h = _gru_step(src_emb_ref[0], h, e_wi, e_wh, e_bi, e_bh)
        h = _gru_step(src_emb_ref[1], h, e_wi, e_wh, e_bi, e_bh)
        hb = h.astype(jnp.bfloat16)
        mean = jnp.dot(hb, wm[...], preferred_element_type=jnp.float32) + bm[...]
        logv = jnp.dot(hb, wv[...], preferred_element_type=jnp.float32) + bv[...]
        kl = -0.5 * jnp.sum(1.0 + logv - mean * mean - jnp.exp(logv), axis=-1)
        kl_out[...] = kl.reshape(1, B)
        h0 = jnp.dot(mean.astype(jnp.bfloat16), wl2h[...],
                     preferred_element_type=jnp.float32) + bl2h[...]
        h_dec[...] = h0
        h_conf[...] = jnp.zeros((B, H), jnp.float32)
        hs = _gru_step(src_emb_ref[0], h0, s_wi, s_wh, s_bi, s_bh)
        nll0 = _logsoftmax_nll(hs, sd_proj, sd_sub[0])
        hs = _gru_step(src_emb_ref[1], hs, s_wi, s_wh, s_bi, s_bh)
        nll1 = _logsoftmax_nll(hs, sd_proj, sd_sub[1])
        tot = jnp.sum((nll0 + nll1).reshape(1, B), axis=-1, keepdims=True)
        sd_out[...] = tot * (0.1 / (2 * B))

    hc = _gru_step(emb_conf_ref[0], h_conf[...], c_wi, c_wh, c_bi, c_bh)
    h_conf[...] = hc
    conf_out[...] = _logsoftmax_nll(hc, conf_proj, src_sub[t]).reshape(1, B, 1)

    hd = _gru_step(trg_emb_ref[0], h_dec[...], d_wi, d_wh, d_bi, d_bh)
    h_dec[...] = hd
    logits = jnp.dot(hd.astype(jnp.bfloat16), proj_head[...],
                     preferred_element_type=jnp.float32)
    col = jax.lax.broadcasted_iota(jnp.int32, (B, L), 1)
    prev = trg_sub[t]
    onehot = (col == prev).astype(jnp.bfloat16)
    mask = jnp.dot(onehot, adj_ref[...], preferred_element_type=jnp.float32)
    mask = jnp.where(t == 0, jnp.float32(1.0), mask)
    mask = jnp.where(col == (L - 2), jnp.float32(1.0), mask)
    m = jnp.max(logits, axis=-1, keepdims=True)
    e = jnp.exp(logits - m) * mask
    s = jnp.sum(e, axis=-1, keepdims=True) + jnp.float32(1e-6) * jnp.exp(-m)
    p = e * pl.reciprocal(s, approx=True)
    p = jnp.where(col == (L - 1), jnp.float32(1.0), p)
    tgt = trg_sub[t + 1]
    picked = jnp.sum(jnp.where(col == tgt, p, 0.0), axis=-1, keepdims=True)
    nll_out[...] = (-jnp.log(picked)).reshape(1, B, 1)


def kernel(road_embedding, projection_head, sd_projection_head, vae_enc_Wi,
           vae_enc_Wh, vae_enc_bi, vae_enc_bh, vae_dec_Wi, vae_dec_Wh,
           vae_dec_bi, vae_dec_bh, vae_sd_dec_Wi, vae_sd_dec_Wh,
           vae_sd_dec_bi, vae_sd_dec_bh, vae_Wm, vae_bm, vae_Wv, vae_bv,
           vae_Wl2h, vae_bl2h, conf_emb, conf_gru_Wi, conf_gru_Wh,
           conf_gru_bi, conf_gru_bh, conf_proj, src, trg, edge_list,
           src_lengths, trg_lengths):
    B, S0 = src.shape
    T = trg.shape[1] - 1
    L, H = road_embedding.shape
    f32 = jnp.float32

    adj = (
        jnp.zeros((L, L), jnp.int32)
        .at[edge_list[0], edge_list[1]]
        .add(1)
        .astype(jnp.bfloat16)
    )
    emb_conf = conf_emb[src.T].astype(f32)
    emb_conf = jnp.pad(emb_conf, ((0, T - S0), (0, 0), (0, 0)))
    trg_t = trg.T.astype(jnp.int32)
    trg_emb = road_embedding[trg_t[:T]].astype(f32)
    cond_src = src[:, 0]
    cond_trg = src[jnp.arange(B), src_lengths - 1]
    sd_idx = jnp.stack([cond_src, cond_trg]).astype(jnp.int32)
    src_emb2 = road_embedding[sd_idx].astype(f32)
    trg_sub = trg_t[:, :, None]
    src_sub = jnp.pad(src.T.astype(jnp.int32), ((0, T - S0), (0, 0)))[:, :, None]
    sd_sub = sd_idx[:, :, None]

    def row(b):
        return b.astype(f32).reshape(1, -1)

    full = lambda shape: pl.BlockSpec(shape, lambda t: (0,) * len(shape))
    step3 = pl.BlockSpec((1, B, H), lambda t: (t, 0, 0))

    grid = (T,)
    in_specs = [
        step3,
        step3,
        full((2, B, H)),
        full((H, 3 * H)), full((H, 3 * H)), full((1, 3 * H)), full((1, 3 * H)),
        full((H, 3 * H)), full((H, 3 * H)), full((1, 3 * H)), full((1, 3 * H)),
        full((H, 3 * H)), full((H, 3 * H)), full((1, 3 * H)), full((1, 3 * H)),
        full((H, 3 * H)), full((H, 3 * H)), full((1, 3 * H)), full((1, 3 * H)),
        full((H, H)), full((1, H)), full((H, H)), full((1, H)),
        full((H, H)), full((1, H)),
        full((H, L)), full((H, L)), full((H, L)),
        full((L, L)),
        full((T + 1, B, 1)),
        full((T, B, 1)),
        full((2, B, 1)),
    ]
    out_specs = [
        pl.BlockSpec((1, B, 1), lambda t: (t, 0, 0)),
        pl.BlockSpec((1, B, 1), lambda t: (t, 0, 0)),
        pl.BlockSpec((1, B), lambda t: (0, 0)),
        pl.BlockSpec((1, 1), lambda t: (0, 0)),
    ]
    out_shape = [
        jax.ShapeDtypeStruct((T, B, 1), f32),
        jax.ShapeDtypeStruct((T, B, 1), f32),
        jax.ShapeDtypeStruct((1, B), f32),
        jax.ShapeDtypeStruct((1, 1), f32),
    ]

    call = pl.pallas_call(
        _fused_kernel,
        grid=grid,
        in_specs=in_specs,
        out_specs=out_specs,
        out_shape=out_shape,
        scratch_shapes=[
            pltpu.VMEM((B, H), f32),
            pltpu.VMEM((B, H), f32),
        ],
        compiler_params=pltpu.CompilerParams(
            dimension_semantics=("arbitrary",),
            vmem_limit_bytes=56 * 1024 * 1024,
        ),
        name="fused_trajgen",
    )
    bf = jnp.bfloat16
    conf_rows, nll_rows, kl_row, sd_val = call(
        emb_conf, trg_emb, src_emb2,
        conf_gru_Wi.astype(bf), conf_gru_Wh.astype(bf),
        row(conf_gru_bi), row(conf_gru_bh),
        vae_dec_Wi.astype(bf), vae_dec_Wh.astype(bf),
        row(vae_dec_bi), row(vae_dec_bh),
        vae_enc_Wi.astype(bf), vae_enc_Wh.astype(bf),
        row(vae_enc_bi), row(vae_enc_bh),
        vae_sd_dec_Wi.astype(bf), vae_sd_dec_Wh.astype(bf),
        row(vae_sd_dec_bi), row(vae_sd_dec_bh),
        vae_Wm.astype(bf), row(vae_bm), vae_Wv.astype(bf), row(vae_bv),
        vae_Wl2h.astype(bf), row(vae_bl2h),
        conf_proj.astype(bf), projection_head.astype(bf),
        sd_projection_head.astype(bf),
        adj, trg_sub, src_sub, sd_sub,
    )

    nll_loss = nll_rows.reshape(T, B).T
    kl_loss = kl_row.reshape(B)
    confidence = conf_rows.reshape(T, B)[:S0].T
    sd_nll_loss = sd_val.reshape(())
    return nll_loss, kl_loss, confidence, sd_nll_loss


# speedup vs baseline: 1.3858x; 1.3858x over previous
import jax
import jax.numpy as jnp
from jax.experimental import pallas as pl
from jax.experimental.pallas import tpu as pltpu

_C = 8


def _gru_gates(gi, gh, h):
    H = h.shape[1]
    i_r, i_z, i_n = gi[:, :H], gi[:, H : 2 * H], gi[:, 2 * H :]
    h_r, h_z, h_n = gh[:, :H], gh[:, H : 2 * H], gh[:, 2 * H :]
    r = jax.nn.sigmoid(i_r + h_r)
    z = jax.nn.sigmoid(i_z + h_z)
    n = jnp.tanh(i_n + r * h_n)
    return (1.0 - z) * n + z * h


def _gru_step(x, h, wi_ref, wh_ref, bi_ref, bh_ref):
    bf = jnp.bfloat16
    gi = jnp.dot(x.astype(bf), wi_ref[...], preferred_element_type=jnp.float32) + bi_ref[...]
    gh = jnp.dot(h.astype(bf), wh_ref[...], preferred_element_type=jnp.float32) + bh_ref[...]
    return _gru_gates(gi, gh, h)


def _logsoftmax_nll(h, proj_ref, tgt):
    logits = jnp.dot(h.astype(jnp.bfloat16), proj_ref[...],
                     preferred_element_type=jnp.float32)
    m = jnp.max(logits, axis=-1, keepdims=True)
    lse = jnp.log(jnp.sum(jnp.exp(logits - m), axis=-1, keepdims=True)) + m
    logp = logits - lse
    col = jax.lax.broadcasted_iota(jnp.int32, logits.shape, 1)
    picked = jnp.sum(jnp.where(col == tgt, logp, 0.0), axis=-1, keepdims=True)
    return -picked


def _fused_kernel(
    emb_conf_ref,
    trg_emb_ref,
    prev_ref,
    tgt_ref,
    ctg_ref,
    src_emb_ref,
    c_wi, c_wh, c_bi, c_bh,
    d_wi, d_wh, d_bi, d_bh,
    e_wi, e_wh, e_bi, e_bh,
    s_wi, s_wh, s_bi, s_bh,
    wm, bm, wv, bv, wl2h, bl2h,
    conf_proj, proj_head, sd_proj,
    adj_ref,
    sd_sub,
    conf_out,
    nll_out,
    kl_out,
    sd_out,
    h_conf, h_dec,
    gic, gid,
    hsc, hsd,
):
    c = pl.program_id(0)
    B, H = h_conf.shape
    L = adj_ref.shape[0]
    C = _C
    M = C * B
    bf = jnp.bfloat16

    @pl.when(c == 0)
    def _init():
        h = jnp.zeros((B, H), jnp.float32)
        h = _gru_step(src_emb_ref[0], h, e_wi, e_wh, e_bi, e_bh)
        h = _gru_step(src_emb_ref[1], h, e_wi, e_wh, e_bi, e_bh)
        hb = h.astype(bf)
        mean = jnp.dot(hb, wm[...], preferred_element_type=jnp.float32) + bm[...]
        logv = jnp.dot(hb, wv[...], preferred_element_type=jnp.float32) + bv[...]
        kl = -0.5 * jnp.sum(1.0 + logv - mean * mean - jnp.exp(logv), axis=-1)
        kl_out[...] = kl.reshape(1, B)
        h0 = jnp.dot(mean.astype(bf), wl2h[...],
                     preferred_element_type=jnp.float32) + bl2h[...]
        h_dec[...] = h0
        h_conf[...] = jnp.zeros((B, H), jnp.float32)
        hs = _gru_step(src_emb_ref[0], h0, s_wi, s_wh, s_bi, s_bh)
        nll0 = _logsoftmax_nll(hs, sd_proj, sd_sub[0])
        hs = _gru_step(src_emb_ref[1], hs, s_wi, s_wh, s_bi, s_bh)
        nll1 = _logsoftmax_nll(hs, sd_proj, sd_sub[1])
        tot = jnp.sum((nll0 + nll1).reshape(1, B), axis=-1, keepdims=True)
        sd_out[...] = tot * (0.1 / (2 * B))

    xc = emb_conf_ref[...].reshape(M, H).astype(bf)
    gic[...] = jnp.dot(xc, c_wi[...], preferred_element_type=jnp.float32) + c_bi[...]
    xd = trg_emb_ref[...].reshape(M, H).astype(bf)
    gid[...] = jnp.dot(xd, d_wi[...], preferred_element_type=jnp.float32) + d_bi[...]

    hc = h_conf[...]
    hd = h_dec[...]
    for s in range(C):
        rows = slice(s * B, (s + 1) * B)
        ghc = jnp.dot(hc.astype(bf), c_wh[...],
                      preferred_element_type=jnp.float32) + c_bh[...]
        hc = _gru_gates(gic[rows], ghc, hc)
        hsc[rows] = hc
        ghd = jnp.dot(hd.astype(bf), d_wh[...],
                      preferred_element_type=jnp.float32) + d_bh[...]
        hd = _gru_gates(gid[rows], ghd, hd)
        hsd[rows] = hd
    h_conf[...] = hc
    h_dec[...] = hd

    conf_out[...] = _logsoftmax_nll(
        hsc[...], conf_proj, ctg_ref[...].reshape(M, 1)
    ).reshape(C, B, 1)

    logits = jnp.dot(hsd[...].astype(bf), proj_head[...],
                     preferred_element_type=jnp.float32)
    col = jax.lax.broadcasted_iota(jnp.int32, (M, L), 1)
    prev = prev_ref[...].reshape(M, 1)
    onehot = (col == prev).astype(bf)
    mask = jnp.dot(onehot, adj_ref[...], preferred_element_type=jnp.float32)
    row = jax.lax.broadcasted_iota(jnp.int32, (M, 1), 0)
    first = jnp.logical_and(c == 0, row < B)
    mask = jnp.where(first, jnp.float32(1.0), mask)
    mask = jnp.where(col == (L - 2), jnp.float32(1.0), mask)
    m = jnp.max(logits, axis=-1, keepdims=True)
    e = jnp.exp(logits - m) * mask
    sden = jnp.sum(e, axis=-1, keepdims=True) + jnp.float32(1e-6) * jnp.exp(-m)
    p = e * pl.reciprocal(sden, approx=True)
    p = jnp.where(col == (L - 1), jnp.float32(1.0), p)
    tgt = tgt_ref[...].reshape(M, 1)
    picked = jnp.sum(jnp.where(col == tgt, p, 0.0), axis=-1, keepdims=True)
    nll_out[...] = (-jnp.log(picked)).reshape(C, B, 1)


def kernel(road_embedding, projection_head, sd_projection_head, vae_enc_Wi,
           vae_enc_Wh, vae_enc_bi, vae_enc_bh, vae_dec_Wi, vae_dec_Wh,
           vae_dec_bi, vae_dec_bh, vae_sd_dec_Wi, vae_sd_dec_Wh,
           vae_sd_dec_bi, vae_sd_dec_bh, vae_Wm, vae_bm, vae_Wv, vae_bv,
           vae_Wl2h, vae_bl2h, conf_emb, conf_gru_Wi, conf_gru_Wh,
           conf_gru_bi, conf_gru_bh, conf_proj, src, trg, edge_list,
           src_lengths, trg_lengths):
    B, S0 = src.shape
    T = trg.shape[1] - 1
    L, H = road_embedding.shape
    C = _C
    f32 = jnp.float32
    bf = jnp.bfloat16

    adj = (
        jnp.zeros((L, L), jnp.int32)
        .at[edge_list[0], edge_list[1]]
        .add(1)
        .astype(bf)
    )
    emb_conf = conf_emb[src.T].astype(f32)
    emb_conf = jnp.pad(emb_conf, ((0, T - S0), (0, 0), (0, 0)))
    trg_t = trg.T.astype(jnp.int32)
    trg_emb = road_embedding[trg_t[:T]].astype(f32)
    cond_src = src[:, 0]
    cond_trg = src[jnp.arange(B), src_lengths - 1]
    sd_idx = jnp.stack([cond_src, cond_trg]).astype(jnp.int32)
    src_emb2 = road_embedding[sd_idx].astype(f32)
    prev_all = trg_t[:T, :, None]
    tgt_all = trg_t[1:, :, None]
    ctg_all = jnp.pad(src.T.astype(jnp.int32), ((0, T - S0), (0, 0)))[:, :, None]
    sd_sub = sd_idx[:, :, None]

    def rowv(b):
        return b.astype(f32).reshape(1, -1)

    full = lambda shape: pl.BlockSpec(shape, lambda c: (0,) * len(shape))
    stepE = pl.BlockSpec((C, B, H), lambda c: (c, 0, 0))
    stepI = pl.BlockSpec((C, B, 1), lambda c: (c, 0, 0))

    grid = (T // C,)
    in_specs = [
        stepE, stepE, stepI, stepI, stepI,
        full((2, B, H)),
        full((H, 3 * H)), full((H, 3 * H)), full((1, 3 * H)), full((1, 3 * H)),
        full((H, 3 * H)), full((H, 3 * H)), full((1, 3 * H)), full((1, 3 * H)),
        full((H, 3 * H)), full((H, 3 * H)), full((1, 3 * H)), full((1, 3 * H)),
        full((H, 3 * H)), full((H, 3 * H)), full((1, 3 * H)), full((1, 3 * H)),
        full((H, H)), full((1, H)), full((H, H)), full((1, H)),
        full((H, H)), full((1, H)),
        full((H, L)), full((H, L)), full((H, L)),
        full((L, L)),
        full((2, B, 1)),
    ]
    out_specs = [
        stepI,
        stepI,
        pl.BlockSpec((1, B), lambda c: (0, 0)),
        pl.BlockSpec((1, 1), lambda c: (0, 0)),
    ]
    out_shape = [
        jax.ShapeDtypeStruct((T, B, 1), f32),
        jax.ShapeDtypeStruct((T, B, 1), f32),
        jax.ShapeDtypeStruct((1, B), f32),
        jax.ShapeDtypeStruct((1, 1), f32),
    ]

    call = pl.pallas_call(
        _fused_kernel,
        grid=grid,
        in_specs=in_specs,
        out_specs=out_specs,
        out_shape=out_shape,
        scratch_shapes=[
            pltpu.VMEM((B, H), f32),
            pltpu.VMEM((B, H), f32),
            pltpu.VMEM((C * B, 3 * H), f32),
            pltpu.VMEM((C * B, 3 * H), f32),
            pltpu.VMEM((C * B, H), f32),
            pltpu.VMEM((C * B, H), f32),
        ],
        compiler_params=pltpu.CompilerParams(
            dimension_semantics=("arbitrary",),
            vmem_limit_bytes=56 * 1024 * 1024,
        ),
        name="fused_trajgen",
    )
    conf_rows, nll_rows, kl_row, sd_val = call(
        emb_conf, trg_emb, prev_all, tgt_all, ctg_all, src_emb2,
        conf_gru_Wi.astype(bf), conf_gru_Wh.astype(bf),
        rowv(conf_gru_bi), rowv(conf_gru_bh),
        vae_dec_Wi.astype(bf), vae_dec_Wh.astype(bf),
        rowv(vae_dec_bi), rowv(vae_dec_bh),
        vae_enc_Wi.astype(bf), vae_enc_Wh.astype(bf),
        rowv(vae_enc_bi), rowv(vae_enc_bh),
        vae_sd_dec_Wi.astype(bf), vae_sd_dec_Wh.astype(bf),
        rowv(vae_sd_dec_bi), rowv(vae_sd_dec_bh),
        vae_Wm.astype(bf), rowv(vae_bm), vae_Wv.astype(bf), rowv(vae_bv),
        vae_Wl2h.astype(bf), rowv(vae_bl2h),
        conf_proj.astype(bf), projection_head.astype(bf),
        sd_projection_head.astype(bf),
        adj, sd_sub,
    )

    nll_loss = nll_rows.reshape(T, B).T
    kl_loss = kl_row.reshape(B)
    confidence = conf_rows.reshape(T, B)[:S0].T
    sd_nll_loss = sd_val.reshape(())
    return nll_loss, kl_loss, confidence, sd_nll_loss
